# baseline (speedup 1.0000x reference)
import functools

import jax
import jax.numpy as jnp
import numpy as np
from jax import lax
from jax.experimental import pallas as pl
from jax.experimental.pallas import tpu as pltpu
from jax.experimental.pallas import tpu_sc as plsc

NC = 2
NS = 16
NW = NC * NS
C = 64
L = 16


def _prep_body(x_ref, w_ref, o_ref):
    x = x_ref[...]
    o_ref[:, :x.shape[1]] = x.astype(jnp.bfloat16)
    o_ref[:, x.shape[1]:] = jnp.dot(
        x, w_ref[...], preferred_element_type=jnp.float32
    ).astype(jnp.bfloat16)


def _prep(x, w, br):
    rows, d = x.shape
    return pl.pallas_call(
        _prep_body,
        grid=(rows // br,),
        in_specs=[
            pl.BlockSpec((br, d), lambda i: (i, 0)),
            pl.BlockSpec((d, d), lambda i: (0, 0)),
        ],
        out_specs=pl.BlockSpec((br, 2 * d), lambda i: (i, 0)),
        out_shape=jax.ShapeDtypeStruct((rows, 2 * d), jnp.bfloat16),
    )(x, w)


def _final_body(p_ref, w_ref, o_ref):
    acc = p_ref[0] + p_ref[1]
    o_ref[...] = jnp.dot(acc, w_ref[...], preferred_element_type=jnp.float32)


def _final(p, w, br, rows):
    d = p.shape[2]
    return pl.pallas_call(
        _final_body,
        grid=(rows // br,),
        in_specs=[
            pl.BlockSpec((2, br, d), lambda i: (0, i, 0)),
            pl.BlockSpec((d, d), lambda i: (0, 0)),
        ],
        out_specs=pl.BlockSpec((br, d), lambda i: (i, 0)),
        out_shape=jax.ShapeDtypeStruct((rows, d), jnp.float32),
    )(p, w)


def _sc_edges(t1, t2, eflat, wbf, wab, n_node, npad):
    d = t1.shape[1]
    g_total = eflat.shape[0] // (6 * C)
    nch_hi = (g_total + NW - 1) // NW
    n_hi = g_total - (nch_hi - 1) * NW
    rows_per_tile = npad // NS
    dch2 = d // 32

    mesh = plsc.VectorSubcoreMesh(core_axis_name="c", subcore_axis_name="s",
                                  num_cores=NC, num_subcores=NS)

    @functools.partial(
        pl.kernel,
        out_type=jax.ShapeDtypeStruct((NC, npad, d), jnp.float32),
        mesh=mesh,
        compiler_params=pltpu.CompilerParams(needs_layout_passes=False),
        scratch_types=[
            pltpu.VMEM((C * 6,), jnp.int32),
            pltpu.VMEM((C * 6,), jnp.int32),
            pltpu.VMEM((C,), jnp.int32),
            pltpu.VMEM((C,), jnp.int32),
            pltpu.VMEM((C,), jnp.int32),
            pltpu.VMEM((C,), jnp.int32),
            pltpu.VMEM((C,), jnp.int32),
            pltpu.VMEM((C,), jnp.int32),
            pltpu.VMEM((C, 128), jnp.float32),
            pltpu.VMEM((C, 128), jnp.float32),
            pltpu.VMEM((C, 128), jnp.float32),
            pltpu.VMEM((C, 128), jnp.float32),
            pltpu.VMEM((C, 128), jnp.float32),
            pltpu.VMEM((C,), jnp.float32),
            pltpu.VMEM((L * L,), jnp.float32),
            pltpu.VMEM((64,), jnp.float32),
            pltpu.VMEM((144,), jnp.float32),
            pltpu.VMEM_SHARED((npad, d), jnp.float32),
            pltpu.SemaphoreType.DMA,
            pltpu.SemaphoreType.DMA,
            pltpu.SemaphoreType.DMA,
            pltpu.SemaphoreType.DMA,
        ],
    )
    def k(t1_hbm, t2_hbm, ef_hbm, wbf_hbm, wab_hbm, out_hbm,
          i3_0, i3_1, subv_0, subv_1, relv_0, relv_1, os_0, os_1,
          t1b_0, t2b_0, t1b_1, t2b_1, msg,
          albuf, sbuf, wvb, wvec, acc, semA0, semA1, semI, semS):
        cid = lax.axis_index("c")
        sid = lax.axis_index("s")
        wid = cid * NS + sid
        nch = jnp.where(wid < n_hi, nch_hi, nch_hi - 1)
        gstart = wid * nch_hi - jnp.maximum(wid - n_hi, 0)

        i3 = (i3_0, i3_1)
        subv = (subv_0, subv_1)
        relv = (relv_0, relv_1)
        os_ = (os_0, os_1)
        t1b = (t1b_0, t1b_1)
        t2b = (t2b_0, t2b_1)
        semA = (semA0, semA1)

        def zfill(r, _):
            for j in range(d // L):
                msg[r, pl.ds(j * L, L)] = jnp.zeros((L,), jnp.float32)
            return 0

        lax.fori_loop(0, C, zfill, 0)
        for part in range(rows_per_tile // C):
            pltpu.sync_copy(
                msg, acc.at[pl.ds(sid * rows_per_tile + part * C, C)])

        pltpu.sync_copy(wbf_hbm, wvb)
        pltpu.sync_copy(wab_hbm, wvec)
        wv = [plsc.bitcast(wvb[pl.ds(j * L, L)], jnp.bfloat16)
              for j in range(dch2)]
        bias_v = jnp.full((L,), wvec[pl.ds(d, L)][0], jnp.float32)
        ev16 = lax.iota(jnp.int32, L) * L
        ev6 = lax.iota(jnp.int32, L) * 6

        plsc.subcore_barrier()

        def issue_idx(gc, s):
            pltpu.async_copy(ef_hbm.at[pl.ds(gc * (6 * C), 6 * C)], i3[s], semI)

        def drain_idx(s):
            pltpu.make_async_copy(
                ef_hbm.at[pl.ds(0, 6 * C)], i3[s], semI).wait()

        def transpose_subrel(s):
            for i in range(C // L):
                base = ev6 + (6 * L * i)
                subv[s][pl.ds(i * L, L)] = plsc.load_gather(i3[s], [base + 4])
                relv[s][pl.ds(i * L, L)] = plsc.load_gather(i3[s], [base + 2])

        def fixup_obj(s):
            for i in range(C // L):
                v = plsc.load_gather(i3[s], [ev6 + (6 * L * i + 5)])
                os_[s][pl.ds(i * L, L)] = lax.rem(v, jnp.int32(n_node))

        def issue_gathers(s):
            pltpu.async_copy(t1_hbm.at[subv[s]], t1b[s], semA[s])
            pltpu.async_copy(t2_hbm.at[relv[s]], t2b[s], semA[s])

        def drain_gathers(s):
            pltpu.make_async_copy(
                t1_hbm.at[pl.ds(0, C)], t1b[s], semA[s]).wait()
            pltpu.make_async_copy(
                t1_hbm.at[pl.ds(0, C)], t2b[s], semA[s]).wait()

        def drain_scatter():
            pltpu.make_async_copy(
                out_hbm.at[0, pl.ds(0, C)], msg, semS).wait()

        issue_idx(gstart, 0)
        drain_idx(0)
        transpose_subrel(0)
        fixup_obj(0)
        issue_gathers(0)
        issue_idx(gstart + 1, 1)

        def do_chunk(c, s):
            s1 = 1 - s
            t1s, t2s = t1b[s], t2b[s]

            drain_gathers(s)

            @pl.when(c + 1 < nch)
            def _():
                drain_idx(s1)
                transpose_subrel(s1)
                issue_gathers(s1)

            @pl.when(c + 2 < nch)
            def _():
                issue_idx(gstart + c + 2, s)

            def alpha_body(g, _):
                ebase = g * L
                for e16 in range(L):
                    i = ebase + e16
                    sb = jnp.zeros((32,), jnp.bfloat16)
                    for j in range(dch2):
                        a = (plsc.bitcast(
                                t1s[i, pl.ds(d // 2 + j * L, L)],
                                jnp.bfloat16)
                             + plsc.bitcast(
                                t2s[i, pl.ds(d // 2 + j * L, L)],
                                jnp.bfloat16))
                        sb = sb + jnp.maximum(a, jnp.bfloat16(0)) * wv[j]
                    sa, sb2 = plsc.unpack(
                        sb, format=plsc.PackFormat.INTERLEAVED)
                    sbuf[pl.ds(e16 * L, L)] = sa + sb2
                tsum = jnp.zeros((L,), jnp.float32)
                for c16 in range(L):
                    tsum = tsum + plsc.load_gather(sbuf, [ev16 + c16])
                alpha_v = 1.0 / (1.0 + jnp.exp(-(tsum + bias_v)))
                albuf[pl.ds(ebase, L)] = alpha_v
                return 0

            lax.fori_loop(0, C // L, alpha_body, 0)

            @pl.when(c != 0)
            def _():
                drain_scatter()

            @pl.when(c + 1 < nch)
            def _():
                fixup_obj(s1)

            def msg_body(g, _):
                ebase = g * L
                alpha_v = albuf[pl.ds(ebase, L)]
                for e16 in range(L):
                    i = ebase + e16
                    av = jnp.full((L,), alpha_v[e16], jnp.float32)
                    for j in range(dch2):
                        m = (plsc.bitcast(
                                t1s[i, pl.ds(j * L, L)], jnp.bfloat16)
                             + plsc.bitcast(
                                t2s[i, pl.ds(j * L, L)], jnp.bfloat16))
                        ma, mb = plsc.unpack(
                            m, format=plsc.PackFormat.INTERLEAVED)
                        msg[i, pl.ds(j * 32, L)] = av * ma
                        msg[i, pl.ds(j * 32 + L, L)] = av * mb
                return 0

            lax.fori_loop(0, C // L, msg_body, 0)
            pltpu.async_copy(msg, acc.at[os_[s]], semS, add=True)

        def pair_body(p, _):
            do_chunk(2 * p, 0)
            do_chunk(2 * p + 1, 1)
            return 0

        lax.fori_loop(0, nch_hi // 2, pair_body, 0)

        @pl.when(nch % 2 == 1)
        def _():
            do_chunk(nch - 1, 0)

        drain_scatter()
        plsc.subcore_barrier()
        pltpu.sync_copy(
            acc.at[pl.ds(sid * rows_per_tile, rows_per_tile)],
            out_hbm.at[cid, pl.ds(sid * rows_per_tile, rows_per_tile)])

    return k(t1, t2, eflat, wbf, wab)


def kernel(hidden, edges, n_node, old_nodes_new_idx, rela_embed, Ws, Wr,
           w_alpha_w, w_alpha_b, W_h):
    n, d = hidden.shape
    v = rela_embed.shape[0]
    e = edges.shape[0]
    g_total = e // C

    eflat = edges.reshape(-1)

    vpad = ((v + 399) // 400) * 400
    rela_p = jnp.pad(rela_embed, ((0, vpad - v), (0, 0)))

    t1 = _prep(hidden, Ws, 400)
    t2 = _prep(rela_p, Wr, 400)
    t1 = lax.bitcast_convert_type(t1.reshape(n, d, 2), jnp.float32)
    t2 = lax.bitcast_convert_type(t2.reshape(vpad, d, 2), jnp.float32)

    wbf = lax.bitcast_convert_type(
        w_alpha_w.reshape(d).astype(jnp.bfloat16).reshape(d // 2, 2),
        jnp.float32)
    wab = jnp.concatenate([
        jnp.zeros((d,), jnp.float32),
        w_alpha_b.reshape(1),
        jnp.zeros((15,), jnp.float32),
    ])

    perm = np.concatenate([
        np.concatenate([32 * j + 2 * np.arange(16),
                        32 * j + 2 * np.arange(16) + 1])
        for j in range(d // 32)
    ])
    w_h_p = jnp.take(W_h, jnp.asarray(perm), axis=0)

    npad = ((n + NS * 64 - 1) // (NS * 64)) * (NS * 64)
    partials = _sc_edges(t1, t2, eflat, wbf, wab, n, npad)
    return _final(partials, w_h_p, 400, n)


# speedup vs baseline: 1.1652x; 1.1652x over previous
import functools

import jax
import jax.numpy as jnp
import numpy as np
from jax import lax
from jax.experimental import pallas as pl
from jax.experimental.pallas import tpu as pltpu
from jax.experimental.pallas import tpu_sc as plsc

NC = 2
NS = 16
NW = NC * NS
C = 64
L = 16


def _prep_body(x_ref, w_ref, o_ref):
    x = x_ref[...]
    o_ref[:, :x.shape[1]] = x.astype(jnp.bfloat16)
    o_ref[:, x.shape[1]:] = jnp.dot(
        x, w_ref[...], preferred_element_type=jnp.float32
    ).astype(jnp.bfloat16)


def _prep(x, w, br):
    rows, d = x.shape
    return pl.pallas_call(
        _prep_body,
        grid=(rows // br,),
        in_specs=[
            pl.BlockSpec((br, d), lambda i: (i, 0)),
            pl.BlockSpec((d, d), lambda i: (0, 0)),
        ],
        out_specs=pl.BlockSpec((br, 2 * d), lambda i: (i, 0)),
        out_shape=jax.ShapeDtypeStruct((rows, 2 * d), jnp.bfloat16),
    )(x, w)


def _final_body(p_ref, w_ref, o_ref):
    acc = p_ref[0] + p_ref[1]
    o_ref[...] = jnp.dot(acc, w_ref[...], preferred_element_type=jnp.float32)


def _final(p, w, br, rows):
    d = p.shape[2]
    return pl.pallas_call(
        _final_body,
        grid=(rows // br,),
        in_specs=[
            pl.BlockSpec((2, br, d), lambda i: (0, i, 0)),
            pl.BlockSpec((d, d), lambda i: (0, 0)),
        ],
        out_specs=pl.BlockSpec((br, d), lambda i: (i, 0)),
        out_shape=jax.ShapeDtypeStruct((rows, d), jnp.float32),
    )(p, w)


def _sc_edges(t1, t2, idx3, wbf, wab, n_node, npad):
    d = t1.shape[1]
    g_total = idx3.shape[0]
    nch_hi = (g_total + NW - 1) // NW
    n_hi = g_total - (nch_hi - 1) * NW
    rows_per_tile = npad // NS
    dch2 = d // 32

    mesh = plsc.VectorSubcoreMesh(core_axis_name="c", subcore_axis_name="s",
                                  num_cores=NC, num_subcores=NS)

    @functools.partial(
        pl.kernel,
        out_type=jax.ShapeDtypeStruct((NC, npad, d), jnp.float32),
        mesh=mesh,
        compiler_params=pltpu.CompilerParams(needs_layout_passes=False),
        scratch_types=[
            pltpu.VMEM((3, 128), jnp.int32),
            pltpu.VMEM((3, 128), jnp.int32),
            pltpu.VMEM((C,), jnp.int32),
            pltpu.VMEM((C,), jnp.int32),
            pltpu.VMEM((C,), jnp.int32),
            pltpu.VMEM((C,), jnp.int32),
            pltpu.VMEM((C, 128), jnp.float32),
            pltpu.VMEM((C, 128), jnp.float32),
            pltpu.VMEM((C, 128), jnp.float32),
            pltpu.VMEM((C, 128), jnp.float32),
            pltpu.VMEM((C, 128), jnp.float32),
            pltpu.VMEM((C,), jnp.float32),
            pltpu.VMEM((L * L,), jnp.float32),
            pltpu.VMEM((64,), jnp.float32),
            pltpu.VMEM((144,), jnp.float32),
            pltpu.VMEM_SHARED((npad, d), jnp.float32),
            pltpu.SemaphoreType.DMA,
            pltpu.SemaphoreType.DMA,
            pltpu.SemaphoreType.DMA,
            pltpu.SemaphoreType.DMA,
        ],
    )
    def k(t1_hbm, t2_hbm, idx3_hbm, wbf_hbm, wab_hbm, out_hbm,
          i3_0, i3_1, op_0, op_1, os_0, os_1,
          t1b_0, t2b_0, t1b_1, t2b_1, msg,
          albuf, sbuf, wvb, wvec, acc, semA0, semA1, semI, semS):
        cid = lax.axis_index("c")
        sid = lax.axis_index("s")
        wid = cid * NS + sid
        nch = jnp.where(wid < n_hi, nch_hi, nch_hi - 1)
        gstart = wid * nch_hi - jnp.maximum(wid - n_hi, 0)

        i3 = (i3_0, i3_1)
        op = (op_0, op_1)
        os_ = (os_0, os_1)
        t1b = (t1b_0, t1b_1)
        t2b = (t2b_0, t2b_1)
        semA = (semA0, semA1)

        def zfill(r, _):
            for j in range(d // L):
                msg[r, pl.ds(j * L, L)] = jnp.zeros((L,), jnp.float32)
            return 0

        lax.fori_loop(0, C, zfill, 0)
        for part in range(rows_per_tile // C):
            pltpu.sync_copy(
                msg, acc.at[pl.ds(sid * rows_per_tile + part * C, C)])

        pltpu.sync_copy(wbf_hbm, wvb)
        pltpu.sync_copy(wab_hbm, wvec)
        wv = [plsc.bitcast(wvb[pl.ds(j * L, L)], jnp.bfloat16)
              for j in range(dch2)]
        bias_v = jnp.full((L,), wvec[pl.ds(d, L)][0], jnp.float32)
        ev16 = lax.iota(jnp.int32, L) * L

        plsc.subcore_barrier()

        def issue_idx(gc, s):
            pltpu.async_copy(idx3_hbm.at[gc], i3[s], semI)

        def drain_idx(s):
            pltpu.make_async_copy(idx3_hbm.at[0], i3[s], semI).wait()

        def fixup_obj(s):
            for i in range(C // L):
                v = i3[s][2, pl.ds(i * L, L)]
                os_[s][pl.ds(i * L, L)] = lax.rem(v, jnp.int32(n_node))

        def issue_gathers(s):
            i_sub = i3[s].at[0, pl.ds(0, C)]
            i_rel = i3[s].at[1, pl.ds(0, C)]
            pltpu.async_copy(t1_hbm.at[i_sub], t1b[s], semA[s])
            pltpu.async_copy(t2_hbm.at[i_rel], t2b[s], semA[s])

        def drain_gathers(s):
            pltpu.make_async_copy(
                t1_hbm.at[pl.ds(0, C)], t1b[s], semA[s]).wait()
            pltpu.make_async_copy(
                t1_hbm.at[pl.ds(0, C)], t2b[s], semA[s]).wait()

        def drain_scatter():
            pltpu.make_async_copy(
                out_hbm.at[0, pl.ds(0, C)], msg, semS).wait()

        issue_idx(gstart, 0)
        drain_idx(0)
        fixup_obj(0)
        issue_gathers(0)
        issue_idx(gstart + 1, 1)

        def do_chunk(c, s):
            s1 = 1 - s
            t1s, t2s = t1b[s], t2b[s]

            drain_gathers(s)

            @pl.when(c + 1 < nch)
            def _():
                drain_idx(s1)
                issue_gathers(s1)

            @pl.when(c + 2 < nch)
            def _():
                issue_idx(gstart + c + 2, s)

            def alpha_body(g, _):
                ebase = g * L
                for e16 in range(L):
                    i = ebase + e16
                    sb = jnp.zeros((32,), jnp.bfloat16)
                    for j in range(dch2):
                        a = (plsc.bitcast(
                                t1s[i, pl.ds(d // 2 + j * L, L)],
                                jnp.bfloat16)
                             + plsc.bitcast(
                                t2s[i, pl.ds(d // 2 + j * L, L)],
                                jnp.bfloat16))
                        sb = sb + jnp.maximum(a, jnp.bfloat16(0)) * wv[j]
                    sa, sb2 = plsc.unpack(
                        sb, format=plsc.PackFormat.INTERLEAVED)
                    sbuf[pl.ds(e16 * L, L)] = sa + sb2
                tsum = jnp.zeros((L,), jnp.float32)
                for c16 in range(L):
                    tsum = tsum + plsc.load_gather(sbuf, [ev16 + c16])
                alpha_v = 1.0 / (1.0 + jnp.exp(-(tsum + bias_v)))
                albuf[pl.ds(ebase, L)] = alpha_v
                return 0

            lax.fori_loop(0, C // L, alpha_body, 0)

            @pl.when(c != 0)
            def _():
                drain_scatter()

            @pl.when(c + 1 < nch)
            def _():
                fixup_obj(s1)

            def msg_body(g, _):
                ebase = g * L
                alpha_v = albuf[pl.ds(ebase, L)]
                for e16 in range(L):
                    i = ebase + e16
                    av = jnp.full((L,), alpha_v[e16], jnp.float32)
                    for j in range(dch2):
                        m = (plsc.bitcast(
                                t1s[i, pl.ds(j * L, L)], jnp.bfloat16)
                             + plsc.bitcast(
                                t2s[i, pl.ds(j * L, L)], jnp.bfloat16))
                        ma, mb = plsc.unpack(
                            m, format=plsc.PackFormat.INTERLEAVED)
                        msg[i, pl.ds(j * 32, L)] = av * ma
                        msg[i, pl.ds(j * 32 + L, L)] = av * mb
                return 0

            lax.fori_loop(0, C // L, msg_body, 0)
            pltpu.async_copy(msg, acc.at[os_[s]], semS, add=True)

        def pair_body(p, _):
            do_chunk(2 * p, 0)
            do_chunk(2 * p + 1, 1)
            return 0

        lax.fori_loop(0, nch_hi // 2, pair_body, 0)

        @pl.when(nch % 2 == 1)
        def _():
            do_chunk(nch - 1, 0)

        drain_scatter()
        plsc.subcore_barrier()
        pltpu.sync_copy(
            acc.at[pl.ds(sid * rows_per_tile, rows_per_tile)],
            out_hbm.at[cid, pl.ds(sid * rows_per_tile, rows_per_tile)])

    return k(t1, t2, idx3, wbf, wab)


def kernel(hidden, edges, n_node, old_nodes_new_idx, rela_embed, Ws, Wr,
           w_alpha_w, w_alpha_b, W_h):
    n, d = hidden.shape
    v = rela_embed.shape[0]
    e = edges.shape[0]
    g_total = e // C

    cols = jnp.stack([edges[:, 4], edges[:, 2], edges[:, 5]])
    idx3 = jnp.transpose(cols.reshape(3, g_total, C), (1, 0, 2))
    idx3 = jnp.pad(idx3, ((0, 0), (0, 0), (0, 128 - C)))

    vpad = ((v + 399) // 400) * 400
    rela_p = jnp.pad(rela_embed, ((0, vpad - v), (0, 0)))

    t1 = _prep(hidden, Ws, 400)
    t2 = _prep(rela_p, Wr, 400)
    t1 = lax.bitcast_convert_type(t1.reshape(n, d, 2), jnp.float32)
    t2 = lax.bitcast_convert_type(t2.reshape(vpad, d, 2), jnp.float32)

    wbf = lax.bitcast_convert_type(
        w_alpha_w.reshape(d).astype(jnp.bfloat16).reshape(d // 2, 2),
        jnp.float32)
    wab = jnp.concatenate([
        jnp.zeros((d,), jnp.float32),
        w_alpha_b.reshape(1),
        jnp.zeros((15,), jnp.float32),
    ])

    perm = np.concatenate([
        np.concatenate([32 * j + 2 * np.arange(16),
                        32 * j + 2 * np.arange(16) + 1])
        for j in range(d // 32)
    ])
    w_h_p = jnp.take(W_h, jnp.asarray(perm), axis=0)

    npad = ((n + NS * 64 - 1) // (NS * 64)) * (NS * 64)
    partials = _sc_edges(t1, t2, idx3, wbf, wab, n, npad)
    return _final(partials, w_h_p, 400, n)
